# baseline (speedup 1.0000x reference)
import functools

import jax
import jax.numpy as jnp
from jax import lax
from jax.experimental import pallas as pl
from jax.experimental.pallas import tpu as pltpu
from jax.experimental.pallas import tpu_sc as plsc

_TBLK = 256
_KBLK = 2048


def _dist_argmin_body(x_ref, xsq_ref, w_ref, wsq_ref, idx_ref):
    K = w_ref.shape[0]
    x = x_ref[...]
    xsq = xsq_ref[...]
    best_d = jnp.full((_TBLK, 1), jnp.inf, dtype=jnp.float32)
    best_i = jnp.zeros((_TBLK, 1), dtype=jnp.int32)
    for kb in range(K // _KBLK):
        w = w_ref[pl.ds(kb * _KBLK, _KBLK), :]
        wsq = wsq_ref[:, pl.ds(kb * _KBLK, _KBLK)]
        mm = lax.dot_general(x, w, (((1,), (1,)), ((), ())),
                             preferred_element_type=jnp.float32)
        d2 = (xsq - 2.0 * mm) + wsq
        dist = jnp.sqrt(jnp.maximum(d2, 0.0))
        m = jnp.min(dist, axis=1, keepdims=True)
        cols = lax.broadcasted_iota(jnp.int32, (_TBLK, _KBLK), 1)
        ii = jnp.min(jnp.where(dist == m, cols, K), axis=1, keepdims=True)
        cand = ii.astype(jnp.int32) + (kb * _KBLK)
        better = m < best_d
        best_d = jnp.where(better, m, best_d)
        best_i = jnp.where(better, cand, best_i)
    idx_ref[...] = best_i


@functools.lru_cache(maxsize=None)
def _make_dist_argmin(n, k, c):
    return pl.pallas_call(
        _dist_argmin_body,
        grid=(n // _TBLK,),
        in_specs=[
            pl.BlockSpec((_TBLK, c), lambda i: (i, 0)),
            pl.BlockSpec((_TBLK, 1), lambda i: (i, 0)),
            pl.BlockSpec((k, c), lambda i: (0, 0)),
            pl.BlockSpec((1, k), lambda i: (0, 0)),
        ],
        out_specs=pl.BlockSpec((_TBLK, 1), lambda i: (i, 0)),
        out_shape=jax.ShapeDtypeStruct((n, 1), jnp.int32),
    )


@functools.lru_cache(maxsize=None)
def _make_sc_gather(n, k, c):
    info = plsc.get_sparse_core_info()
    nw = info.num_cores * info.num_subcores
    bpw = n // nw
    mesh = plsc.VectorSubcoreMesh(core_axis_name="c", subcore_axis_name="s")

    @functools.partial(
        pl.kernel,
        mesh=mesh,
        compiler_params=pltpu.CompilerParams(use_tc_tiling_on_sc=False),
        out_type=jax.ShapeDtypeStruct((n, c), jnp.float32),
        scratch_types=[
            pltpu.VMEM((bpw,), jnp.int32),
            pltpu.VMEM((bpw, c), jnp.float32),
            pltpu.SemaphoreType.DMA,
        ],
    )
    def gather(table_hbm, idx_hbm, out_hbm, idx_v, rows_v, sem):
        wid = lax.axis_index("s") * info.num_cores + lax.axis_index("c")
        base = wid * bpw
        pltpu.sync_copy(idx_hbm.at[pl.ds(base, bpw)], idx_v)
        pltpu.async_copy(table_hbm.at[idx_v], rows_v, sem).wait()
        pltpu.sync_copy(rows_v, out_hbm.at[pl.ds(base, bpw)])

    return gather


def kernel(z_e, W):
    z = jnp.transpose(z_e, (0, 2, 3, 1))
    B, Hh, Ww, C = z.shape
    flat = z.reshape(-1, C)
    N = flat.shape[0]
    K = W.shape[0]
    xsq = jnp.sum(flat * flat, axis=1, keepdims=True)
    wsq = jnp.sum(W * W, axis=1)[None, :]
    indices = _make_dist_argmin(N, K, C)(flat, xsq, W, wsq)[:, 0]
    z_q = _make_sc_gather(N, K, C)(W, indices)
    z_q4 = z_q.reshape(z.shape)
    z_q_st = z + lax.stop_gradient(z_q4 - z)
    return (
        jnp.transpose(z_q_st, (0, 3, 1, 2)),
        jnp.transpose(z_q4, (0, 3, 1, 2)),
        indices.reshape(B, Hh * Ww),
    )


# speedup vs baseline: 1.0583x; 1.0583x over previous
import functools

import jax
import jax.numpy as jnp
from jax import lax
from jax.experimental import pallas as pl
from jax.experimental.pallas import tpu as pltpu
from jax.experimental.pallas import tpu_sc as plsc

_TBLK = 512
_KBLK = 4096


def _dist_argmin_body(x_ref, xsq_ref, w_ref, wsq_ref, idx_ref):
    K = w_ref.shape[0]
    x = x_ref[...]
    xsq = xsq_ref[...]
    best_d = jnp.full((_TBLK, 1), jnp.inf, dtype=jnp.float32)
    best_i = jnp.zeros((_TBLK, 1), dtype=jnp.int32)
    for kb in range(K // _KBLK):
        w = w_ref[pl.ds(kb * _KBLK, _KBLK), :]
        wsq = wsq_ref[:, pl.ds(kb * _KBLK, _KBLK)]
        mm = lax.dot_general(x, w, (((1,), (1,)), ((), ())),
                             preferred_element_type=jnp.float32)
        d2 = (xsq - 2.0 * mm) + wsq
        dist = jnp.sqrt(jnp.maximum(d2, 0.0))
        m = jnp.min(dist, axis=1, keepdims=True)
        cols = lax.broadcasted_iota(jnp.int32, (_TBLK, _KBLK), 1)
        ii = jnp.min(jnp.where(dist == m, cols, K), axis=1, keepdims=True)
        cand = ii.astype(jnp.int32) + (kb * _KBLK)
        better = m < best_d
        best_d = jnp.where(better, m, best_d)
        best_i = jnp.where(better, cand, best_i)
    idx_ref[...] = best_i


@functools.lru_cache(maxsize=None)
def _make_dist_argmin(n, k, c):
    return pl.pallas_call(
        _dist_argmin_body,
        grid=(n // _TBLK,),
        in_specs=[
            pl.BlockSpec((_TBLK, c), lambda i: (i, 0)),
            pl.BlockSpec((_TBLK, 1), lambda i: (i, 0)),
            pl.BlockSpec((k, c), lambda i: (0, 0)),
            pl.BlockSpec((1, k), lambda i: (0, 0)),
        ],
        out_specs=pl.BlockSpec((_TBLK, 1), lambda i: (i, 0)),
        out_shape=jax.ShapeDtypeStruct((n, 1), jnp.int32),
    )


@functools.lru_cache(maxsize=None)
def _make_sc_gather(n, k, c):
    info = plsc.get_sparse_core_info()
    nw = info.num_cores * info.num_subcores
    bpw = n // nw
    mesh = plsc.VectorSubcoreMesh(core_axis_name="c", subcore_axis_name="s")

    @functools.partial(
        pl.kernel,
        mesh=mesh,
        compiler_params=pltpu.CompilerParams(use_tc_tiling_on_sc=False),
        out_type=jax.ShapeDtypeStruct((n, c), jnp.float32),
        scratch_types=[
            pltpu.VMEM((bpw,), jnp.int32),
            pltpu.VMEM((bpw, c), jnp.float32),
            pltpu.SemaphoreType.DMA,
        ],
    )
    def gather(table_hbm, idx_hbm, out_hbm, idx_v, rows_v, sem):
        wid = lax.axis_index("s") * info.num_cores + lax.axis_index("c")
        base = wid * bpw
        pltpu.sync_copy(idx_hbm.at[pl.ds(base, bpw)], idx_v)
        pltpu.async_copy(table_hbm.at[idx_v], rows_v, sem).wait()
        pltpu.sync_copy(rows_v, out_hbm.at[pl.ds(base, bpw)])

    return gather


def kernel(z_e, W):
    z = jnp.transpose(z_e, (0, 2, 3, 1))
    B, Hh, Ww, C = z.shape
    flat = z.reshape(-1, C)
    N = flat.shape[0]
    K = W.shape[0]
    xsq = jnp.sum(flat * flat, axis=1, keepdims=True)
    wsq = jnp.sum(W * W, axis=1)[None, :]
    indices = _make_dist_argmin(N, K, C)(flat, xsq, W, wsq)[:, 0]
    z_q = _make_sc_gather(N, K, C)(W, indices)
    z_q4 = z_q.reshape(z.shape)
    z_q_st = z + lax.stop_gradient(z_q4 - z)
    return (
        jnp.transpose(z_q_st, (0, 3, 1, 2)),
        jnp.transpose(z_q4, (0, 3, 1, 2)),
        indices.reshape(B, Hh * Ww),
    )


# speedup vs baseline: 1.0851x; 1.0253x over previous
import functools

import jax
import jax.numpy as jnp
from jax import lax
from jax.experimental import pallas as pl
from jax.experimental.pallas import tpu as pltpu
from jax.experimental.pallas import tpu_sc as plsc

_TBLK = 512
_KBLK = 4096


def _dist_argmin_body(x_ref, xsq_ref, w2_ref, wsq_ref, idx_ref):
    K = w2_ref.shape[0]
    x = x_ref[...]
    xsq = xsq_ref[...]
    best_d = jnp.full((_TBLK, 1), jnp.inf, dtype=jnp.float32)
    best_i = jnp.zeros((_TBLK, 1), dtype=jnp.int32)
    for kb in range(K // _KBLK):
        w2 = w2_ref[pl.ds(kb * _KBLK, _KBLK), :]
        wsq = wsq_ref[:, pl.ds(kb * _KBLK, _KBLK)]
        mm2 = lax.dot_general(x, w2, (((1,), (1,)), ((), ())),
                              preferred_element_type=jnp.float32)
        d2 = (xsq + mm2) + wsq
        dist = jnp.sqrt(jnp.maximum(d2, 0.0))
        m = jnp.min(dist, axis=1, keepdims=True)
        cols = lax.broadcasted_iota(jnp.int32, (_TBLK, _KBLK), 1)
        ii = jnp.min(jnp.where(dist == m, cols, K), axis=1, keepdims=True)
        cand = ii.astype(jnp.int32) + (kb * _KBLK)
        better = m < best_d
        best_d = jnp.where(better, m, best_d)
        best_i = jnp.where(better, cand, best_i)
    idx_ref[...] = best_i


@functools.lru_cache(maxsize=None)
def _make_dist_argmin(n, k, c):
    return pl.pallas_call(
        _dist_argmin_body,
        grid=(n // _TBLK,),
        in_specs=[
            pl.BlockSpec((_TBLK, c), lambda i: (i, 0)),
            pl.BlockSpec((_TBLK, 1), lambda i: (i, 0)),
            pl.BlockSpec((k, c), lambda i: (0, 0)),
            pl.BlockSpec((1, k), lambda i: (0, 0)),
        ],
        out_specs=pl.BlockSpec((_TBLK, 1), lambda i: (i, 0)),
        out_shape=jax.ShapeDtypeStruct((n, 1), jnp.int32),
    )


@functools.lru_cache(maxsize=None)
def _make_sc_gather(n, k, c):
    info = plsc.get_sparse_core_info()
    nw = info.num_cores * info.num_subcores
    bpw = n // nw
    mesh = plsc.VectorSubcoreMesh(core_axis_name="c", subcore_axis_name="s")

    @functools.partial(
        pl.kernel,
        mesh=mesh,
        compiler_params=pltpu.CompilerParams(use_tc_tiling_on_sc=False),
        out_type=jax.ShapeDtypeStruct((n, c), jnp.float32),
        scratch_types=[
            pltpu.VMEM((bpw,), jnp.int32),
            pltpu.VMEM((bpw, c), jnp.float32),
            pltpu.SemaphoreType.DMA,
        ],
    )
    def gather(table_hbm, idx_hbm, out_hbm, idx_v, rows_v, sem):
        wid = lax.axis_index("s") * info.num_cores + lax.axis_index("c")
        base = wid * bpw
        pltpu.sync_copy(idx_hbm.at[pl.ds(base, bpw)], idx_v)
        pltpu.async_copy(table_hbm.at[idx_v], rows_v, sem).wait()
        pltpu.sync_copy(rows_v, out_hbm.at[pl.ds(base, bpw)])

    return gather


def kernel(z_e, W):
    z = jnp.transpose(z_e, (0, 2, 3, 1))
    B, Hh, Ww, C = z.shape
    flat = z.reshape(-1, C)
    N = flat.shape[0]
    K = W.shape[0]
    xsq = jnp.sum(flat * flat, axis=1, keepdims=True)
    wsq = jnp.sum(W * W, axis=1)[None, :]
    indices = _make_dist_argmin(N, K, C)(flat, xsq, -2.0 * W, wsq)[:, 0]
    z_q = _make_sc_gather(N, K, C)(W, indices)
    z_q4 = z_q.reshape(z.shape)
    z_q_st = z + lax.stop_gradient(z_q4 - z)
    return (
        jnp.transpose(z_q_st, (0, 3, 1, 2)),
        jnp.transpose(z_q4, (0, 3, 1, 2)),
        indices.reshape(B, Hh * Ww),
    )


# speedup vs baseline: 1.0875x; 1.0022x over previous
import functools

import jax
import jax.numpy as jnp
from jax import lax
from jax.experimental import pallas as pl
from jax.experimental.pallas import tpu as pltpu
from jax.experimental.pallas import tpu_sc as plsc

_TBLK = 512
_KBLK = 8192


def _dist_argmin_body(x_ref, xsq_ref, w2_ref, wsq_ref, idx_ref):
    K = w2_ref.shape[0]
    x = x_ref[...]
    xsq = xsq_ref[...]
    best_d = jnp.full((_TBLK, 1), jnp.inf, dtype=jnp.float32)
    best_i = jnp.zeros((_TBLK, 1), dtype=jnp.int32)
    for kb in range(K // _KBLK):
        w2 = w2_ref[pl.ds(kb * _KBLK, _KBLK), :]
        wsq = wsq_ref[:, pl.ds(kb * _KBLK, _KBLK)]
        mm2 = lax.dot_general(x, w2, (((1,), (1,)), ((), ())),
                              preferred_element_type=jnp.float32)
        d2 = (xsq + mm2) + wsq
        dist = jnp.sqrt(jnp.maximum(d2, 0.0))
        m = jnp.min(dist, axis=1, keepdims=True)
        cols = lax.broadcasted_iota(jnp.int32, (_TBLK, _KBLK), 1)
        ii = jnp.min(jnp.where(dist == m, cols, K), axis=1, keepdims=True)
        cand = ii.astype(jnp.int32) + (kb * _KBLK)
        better = m < best_d
        best_d = jnp.where(better, m, best_d)
        best_i = jnp.where(better, cand, best_i)
    idx_ref[...] = best_i


@functools.lru_cache(maxsize=None)
def _make_dist_argmin(n, k, c):
    return pl.pallas_call(
        _dist_argmin_body,
        grid=(n // _TBLK,),
        in_specs=[
            pl.BlockSpec((_TBLK, c), lambda i: (i, 0)),
            pl.BlockSpec((_TBLK, 1), lambda i: (i, 0)),
            pl.BlockSpec((k, c), lambda i: (0, 0)),
            pl.BlockSpec((1, k), lambda i: (0, 0)),
        ],
        out_specs=pl.BlockSpec((_TBLK, 1), lambda i: (i, 0)),
        out_shape=jax.ShapeDtypeStruct((n, 1), jnp.int32),
    )


@functools.lru_cache(maxsize=None)
def _make_sc_gather(n, k, c):
    info = plsc.get_sparse_core_info()
    nw = info.num_cores * info.num_subcores
    bpw = n // nw
    mesh = plsc.VectorSubcoreMesh(core_axis_name="c", subcore_axis_name="s")

    @functools.partial(
        pl.kernel,
        mesh=mesh,
        compiler_params=pltpu.CompilerParams(use_tc_tiling_on_sc=False),
        out_type=jax.ShapeDtypeStruct((n, c), jnp.float32),
        scratch_types=[
            pltpu.VMEM((bpw,), jnp.int32),
            pltpu.VMEM((bpw, c), jnp.float32),
            pltpu.SemaphoreType.DMA,
        ],
    )
    def gather(table_hbm, idx_hbm, out_hbm, idx_v, rows_v, sem):
        wid = lax.axis_index("s") * info.num_cores + lax.axis_index("c")
        base = wid * bpw
        pltpu.sync_copy(idx_hbm.at[pl.ds(base, bpw)], idx_v)
        pltpu.async_copy(table_hbm.at[idx_v], rows_v, sem).wait()
        pltpu.sync_copy(rows_v, out_hbm.at[pl.ds(base, bpw)])

    return gather


def kernel(z_e, W):
    z = jnp.transpose(z_e, (0, 2, 3, 1))
    B, Hh, Ww, C = z.shape
    flat = z.reshape(-1, C)
    N = flat.shape[0]
    K = W.shape[0]
    xsq = jnp.sum(flat * flat, axis=1, keepdims=True)
    wsq = jnp.sum(W * W, axis=1)[None, :]
    indices = _make_dist_argmin(N, K, C)(flat, xsq, -2.0 * W, wsq)[:, 0]
    z_q = _make_sc_gather(N, K, C)(W, indices)
    z_q4 = z_q.reshape(z.shape)
    z_q_st = z + lax.stop_gradient(z_q4 - z)
    return (
        jnp.transpose(z_q_st, (0, 3, 1, 2)),
        jnp.transpose(z_q4, (0, 3, 1, 2)),
        indices.reshape(B, Hh * Ww),
    )


# speedup vs baseline: 1.5870x; 1.4593x over previous
import functools

import jax
import jax.numpy as jnp
from jax import lax
from jax.experimental import pallas as pl
from jax.experimental.pallas import tpu as pltpu
from jax.experimental.pallas import tpu_sc as plsc

_TBLK = 512
_KBLK = 8192


def _dist_argmin_body(x_ref, xsq_ref, w2_ref, wsq_ref, idx_ref):
    K = w2_ref.shape[0]
    x = x_ref[...]
    xsq = xsq_ref[...]
    best_d = jnp.full((_TBLK, 1), jnp.inf, dtype=jnp.float32)
    best_i = jnp.zeros((_TBLK, 1), dtype=jnp.int32)
    for kb in range(K // _KBLK):
        w2 = w2_ref[pl.ds(kb * _KBLK, _KBLK), :]
        wsq = wsq_ref[:, pl.ds(kb * _KBLK, _KBLK)]
        mm2 = lax.dot_general(x, w2, (((1,), (1,)), ((), ())),
                              preferred_element_type=jnp.float32)
        d2 = (xsq + mm2) + wsq
        m = jnp.min(d2, axis=1, keepdims=True)
        cols = lax.broadcasted_iota(jnp.int32, (_TBLK, _KBLK), 1)
        ii = jnp.min(jnp.where(d2 == m, cols, K), axis=1, keepdims=True)
        cand = ii.astype(jnp.int32) + (kb * _KBLK)
        better = m < best_d
        best_d = jnp.where(better, m, best_d)
        best_i = jnp.where(better, cand, best_i)
    idx_ref[...] = best_i


@functools.lru_cache(maxsize=None)
def _make_dist_argmin(n, k, c):
    return pl.pallas_call(
        _dist_argmin_body,
        grid=(n // _TBLK,),
        in_specs=[
            pl.BlockSpec((_TBLK, c), lambda i: (i, 0)),
            pl.BlockSpec((_TBLK, 1), lambda i: (i, 0)),
            pl.BlockSpec((k, c), lambda i: (0, 0)),
            pl.BlockSpec((1, k), lambda i: (0, 0)),
        ],
        out_specs=pl.BlockSpec((_TBLK, 1), lambda i: (i, 0)),
        out_shape=jax.ShapeDtypeStruct((n, 1), jnp.int32),
    )


@functools.lru_cache(maxsize=None)
def _make_sc_gather(n, k, c):
    info = plsc.get_sparse_core_info()
    nw = info.num_cores * info.num_subcores
    bpw = n // nw
    mesh = plsc.VectorSubcoreMesh(core_axis_name="c", subcore_axis_name="s")

    @functools.partial(
        pl.kernel,
        mesh=mesh,
        compiler_params=pltpu.CompilerParams(use_tc_tiling_on_sc=False),
        out_type=jax.ShapeDtypeStruct((n, c), jnp.float32),
        scratch_types=[
            pltpu.VMEM((bpw,), jnp.int32),
            pltpu.VMEM((bpw, c), jnp.float32),
            pltpu.SemaphoreType.DMA,
        ],
    )
    def gather(table_hbm, idx_hbm, out_hbm, idx_v, rows_v, sem):
        wid = lax.axis_index("s") * info.num_cores + lax.axis_index("c")
        base = wid * bpw
        pltpu.sync_copy(idx_hbm.at[pl.ds(base, bpw)], idx_v)
        pltpu.async_copy(table_hbm.at[idx_v], rows_v, sem).wait()
        pltpu.sync_copy(rows_v, out_hbm.at[pl.ds(base, bpw)])

    return gather


def kernel(z_e, W):
    z = jnp.transpose(z_e, (0, 2, 3, 1))
    B, Hh, Ww, C = z.shape
    flat = z.reshape(-1, C)
    N = flat.shape[0]
    K = W.shape[0]
    xsq = jnp.sum(flat * flat, axis=1, keepdims=True)
    wsq = jnp.sum(W * W, axis=1)[None, :]
    indices = _make_dist_argmin(N, K, C)(flat, xsq, -2.0 * W, wsq)[:, 0]
    z_q = _make_sc_gather(N, K, C)(W, indices)
    z_q4 = z_q.reshape(z.shape)
    z_q_st = z + lax.stop_gradient(z_q4 - z)
    return (
        jnp.transpose(z_q_st, (0, 3, 1, 2)),
        jnp.transpose(z_q4, (0, 3, 1, 2)),
        indices.reshape(B, Hh * Ww),
    )


# speedup vs baseline: 1.8760x; 1.1822x over previous
import functools

import jax
import jax.numpy as jnp
from jax import lax
from jax.experimental import pallas as pl
from jax.experimental.pallas import tpu as pltpu
from jax.experimental.pallas import tpu_sc as plsc

_TBLK = 512
_KBLK = 8192


def _dist_argmin_body(x_ref, xsq_ref, w2_ref, wsq_ref, idx_ref):
    x = x_ref[...]
    xsq = xsq_ref[...]
    w2 = w2_ref[...]
    wsq = wsq_ref[...]
    mm2 = lax.dot_general(x, w2, (((1,), (1,)), ((), ())),
                          preferred_element_type=jnp.float32)
    d2 = (xsq + mm2) + wsq
    idx_ref[...] = jnp.argmin(d2, axis=1, keepdims=True).astype(jnp.int32)


@functools.lru_cache(maxsize=None)
def _make_dist_argmin(n, k, c):
    return pl.pallas_call(
        _dist_argmin_body,
        grid=(n // _TBLK,),
        in_specs=[
            pl.BlockSpec((_TBLK, c), lambda i: (i, 0)),
            pl.BlockSpec((_TBLK, 1), lambda i: (i, 0)),
            pl.BlockSpec((k, c), lambda i: (0, 0)),
            pl.BlockSpec((1, k), lambda i: (0, 0)),
        ],
        out_specs=pl.BlockSpec((_TBLK, 1), lambda i: (i, 0)),
        out_shape=jax.ShapeDtypeStruct((n, 1), jnp.int32),
    )


@functools.lru_cache(maxsize=None)
def _make_sc_gather(n, k, c):
    info = plsc.get_sparse_core_info()
    nw = info.num_cores * info.num_subcores
    bpw = n // nw
    mesh = plsc.VectorSubcoreMesh(core_axis_name="c", subcore_axis_name="s")

    @functools.partial(
        pl.kernel,
        mesh=mesh,
        compiler_params=pltpu.CompilerParams(use_tc_tiling_on_sc=False),
        out_type=jax.ShapeDtypeStruct((n, c), jnp.float32),
        scratch_types=[
            pltpu.VMEM((bpw,), jnp.int32),
            pltpu.VMEM((bpw, c), jnp.float32),
            pltpu.SemaphoreType.DMA,
        ],
    )
    def gather(table_hbm, idx_hbm, out_hbm, idx_v, rows_v, sem):
        wid = lax.axis_index("s") * info.num_cores + lax.axis_index("c")
        base = wid * bpw
        pltpu.sync_copy(idx_hbm.at[pl.ds(base, bpw)], idx_v)
        pltpu.async_copy(table_hbm.at[idx_v], rows_v, sem).wait()
        pltpu.sync_copy(rows_v, out_hbm.at[pl.ds(base, bpw)])

    return gather


def kernel(z_e, W):
    z = jnp.transpose(z_e, (0, 2, 3, 1))
    B, Hh, Ww, C = z.shape
    flat = z.reshape(-1, C)
    N = flat.shape[0]
    K = W.shape[0]
    xsq = jnp.sum(flat * flat, axis=1, keepdims=True)
    wsq = jnp.sum(W * W, axis=1)[None, :]
    indices = _make_dist_argmin(N, K, C)(flat, xsq, -2.0 * W, wsq)[:, 0]
    z_q = _make_sc_gather(N, K, C)(W, indices)
    z_q4 = z_q.reshape(z.shape)
    z_q_st = z + lax.stop_gradient(z_q4 - z)
    return (
        jnp.transpose(z_q_st, (0, 3, 1, 2)),
        jnp.transpose(z_q4, (0, 3, 1, 2)),
        indices.reshape(B, Hh * Ww),
    )


# speedup vs baseline: 1.9117x; 1.0190x over previous
import functools

import jax
import jax.numpy as jnp
from jax import lax
from jax.experimental import pallas as pl
from jax.experimental.pallas import tpu as pltpu
from jax.experimental.pallas import tpu_sc as plsc

_TBLK = 1024
_KBLK = 8192


def _dist_argmin_body(x_ref, xsq_ref, w2_ref, wsq_ref, idx_ref):
    x = x_ref[...]
    xsq = xsq_ref[...]
    w2 = w2_ref[...]
    wsq = wsq_ref[...]
    mm2 = lax.dot_general(x, w2, (((1,), (1,)), ((), ())),
                          preferred_element_type=jnp.float32)
    d2 = (xsq + mm2) + wsq
    idx_ref[...] = jnp.argmin(d2, axis=1, keepdims=True).astype(jnp.int32)


@functools.lru_cache(maxsize=None)
def _make_dist_argmin(n, k, c):
    return pl.pallas_call(
        _dist_argmin_body,
        grid=(n // _TBLK,),
        in_specs=[
            pl.BlockSpec((_TBLK, c), lambda i: (i, 0)),
            pl.BlockSpec((_TBLK, 1), lambda i: (i, 0)),
            pl.BlockSpec((k, c), lambda i: (0, 0)),
            pl.BlockSpec((1, k), lambda i: (0, 0)),
        ],
        out_specs=pl.BlockSpec((_TBLK, 1), lambda i: (i, 0)),
        out_shape=jax.ShapeDtypeStruct((n, 1), jnp.int32),
    )


@functools.lru_cache(maxsize=None)
def _make_sc_gather(n, k, c):
    info = plsc.get_sparse_core_info()
    nw = info.num_cores * info.num_subcores
    bpw = n // nw
    mesh = plsc.VectorSubcoreMesh(core_axis_name="c", subcore_axis_name="s")

    @functools.partial(
        pl.kernel,
        mesh=mesh,
        compiler_params=pltpu.CompilerParams(use_tc_tiling_on_sc=False),
        out_type=jax.ShapeDtypeStruct((n, c), jnp.float32),
        scratch_types=[
            pltpu.VMEM((bpw,), jnp.int32),
            pltpu.VMEM((bpw, c), jnp.float32),
            pltpu.SemaphoreType.DMA,
        ],
    )
    def gather(table_hbm, idx_hbm, out_hbm, idx_v, rows_v, sem):
        wid = lax.axis_index("s") * info.num_cores + lax.axis_index("c")
        base = wid * bpw
        pltpu.sync_copy(idx_hbm.at[pl.ds(base, bpw)], idx_v)
        pltpu.async_copy(table_hbm.at[idx_v], rows_v, sem).wait()
        pltpu.sync_copy(rows_v, out_hbm.at[pl.ds(base, bpw)])

    return gather


def kernel(z_e, W):
    z = jnp.transpose(z_e, (0, 2, 3, 1))
    B, Hh, Ww, C = z.shape
    flat = z.reshape(-1, C)
    N = flat.shape[0]
    K = W.shape[0]
    xsq = jnp.sum(flat * flat, axis=1, keepdims=True)
    wsq = jnp.sum(W * W, axis=1)[None, :]
    indices = _make_dist_argmin(N, K, C)(flat, xsq, -2.0 * W, wsq)[:, 0]
    z_q = _make_sc_gather(N, K, C)(W, indices)
    z_q4 = z_q.reshape(z.shape)
    z_q_st = z + lax.stop_gradient(z_q4 - z)
    return (
        jnp.transpose(z_q_st, (0, 3, 1, 2)),
        jnp.transpose(z_q4, (0, 3, 1, 2)),
        indices.reshape(B, Hh * Ww),
    )
